# baseline (speedup 1.0000x reference)
import functools

import jax
import jax.numpy as jnp
from jax import lax
from jax.experimental import pallas as pl
from jax.experimental.pallas import tpu as pltpu
from jax.experimental.pallas import tpu_sc as plsc

_CUTOFF2 = 25.0
_K = 32
_BIG = 1e10
_BIGH = 0.5e10
_INIT = 2e10
_MAXI = 0x7FFFFFFF
_NC = 2
_NS = 16
_LANES = 16


def _iota():
    return lax.iota(jnp.int32, _LANES)


def _shuffle(v, perm):
    dnums = lax.GatherDimensionNumbers(offset_dims=(),
                                       collapsed_slice_dims=(0,),
                                       start_index_map=(0,))
    return lax.gather(v, perm[:, None], dnums, (1,),
                      mode=lax.GatherScatterMode.PROMISE_IN_BOUNDS)


def _bf16_round(x):
    b = plsc.bitcast(x, jnp.int32)
    r = b + 0x7FFF + (lax.shift_right_arithmetic(b, 16) & 1)
    return plsc.bitcast(r & jnp.int32(-65536), jnp.float32)


def _rsqrt_newton(x):
    ib = plsc.bitcast(x, jnp.int32)
    y = plsc.bitcast(jnp.int32(0x5F3759DF) - lax.shift_right_arithmetic(ib, 1),
                     jnp.float32)
    for _ in range(2):
        y = y * (1.5 - 0.5 * x * y * y)
    return y


def _lexlt(va, ia, vb, ib):
    return (va < vb) | ((va == vb) & (ia < ib))


def _ce_cross(a, b, desc):
    va, ia = a
    vb, ib = b
    lt = _lexlt(va, ia, vb, ib)
    keep = ~lt if desc else lt
    return ((jnp.where(keep, va, vb), jnp.where(keep, ia, ib)),
            (jnp.where(keep, vb, va), jnp.where(keep, ib, ia)))


def _ce_intra(p, t, j, k, iot):
    v, ix = p
    perm = iot ^ j
    pv = _shuffle(v, perm)
    pi = _shuffle(ix, perm)
    lt = _lexlt(v, ix, pv, pi)
    keep = lt ^ ((iot & j) != 0) ^ (((t * _LANES + iot) & k) != 0)
    return (jnp.where(keep, v, pv), jnp.where(keep, ix, pi))


def _bitonic_top32(P, iot):
    nv = len(P)
    for k in (2, 4, 8, 16):
        for j in (8, 4, 2, 1):
            if j < k:
                P = [_ce_intra(P[t], t, j, k, iot) for t in range(nv)]
    if nv == 2:
        P[0], P[1] = _ce_cross(P[0], P[1], False)
        for j in (8, 4, 2, 1):
            P = [_ce_intra(P[t], t, j, 32, iot) for t in range(2)]
        return (P[0][1], P[1][1], P[0][0], P[1][0])
    P[0], P[1] = _ce_cross(P[0], P[1], False)
    P[2], P[3] = _ce_cross(P[2], P[3], True)
    for j in (8, 4, 2, 1):
        P = [_ce_intra(P[t], t, j, 32, iot) for t in range(4)]
    P[0], P[2] = _ce_cross(P[0], P[2], False)
    P[1], P[3] = _ce_cross(P[1], P[3], False)
    P[0], P[1] = _ce_cross(P[0], P[1], False)
    for j in (8, 4, 2, 1):
        P = [_ce_intra(P[t], t, j, 64, iot) for t in range(2)] + P[2:]
    return (P[0][1], P[1][1], P[0][0], P[1][0])


def _build(posf, batch, n, rows_per_worker):
    k3 = 3 * _K
    mesh = plsc.VectorSubcoreMesh(core_axis_name="c", subcore_axis_name="s",
                                  num_cores=_NC, num_subcores=_NS)
    out_type = (
        jax.ShapeDtypeStruct((2, n * _K), jnp.int32),
        jax.ShapeDtypeStruct((n * _K,), jnp.float32),
        jax.ShapeDtypeStruct((n * k3,), jnp.float32),
    )
    scratch = (
        pltpu.VMEM((3 * n,), jnp.float32),
        pltpu.VMEM((n,), jnp.int32),
        pltpu.VMEM((n,), jnp.float32),
        pltpu.VMEM((rows_per_worker * _K,), jnp.int32),
        pltpu.VMEM((rows_per_worker * _K,), jnp.int32),
        pltpu.VMEM((rows_per_worker * _K,), jnp.float32),
        pltpu.VMEM((rows_per_worker * k3,), jnp.float32),
    )

    @functools.partial(pl.kernel, out_type=out_type, mesh=mesh,
                       scratch_types=scratch,
                       compiler_params=pltpu.CompilerParams(
                           needs_layout_passes=False))
    def body(posf_hbm, batch_hbm, ei_hbm, wgt_hbm, vec_hbm,
             posf_v, batch_v, d2buf, src_v, dst_v, wgt_v, vec_v):
        wid = lax.axis_index("s") * _NC + lax.axis_index("c")
        rowbase = wid * rows_per_worker
        pltpu.sync_copy(posf_hbm, posf_v)
        pltpu.sync_copy(batch_hbm, batch_v)

        iot = _iota()
        bigv = jnp.full((_LANES,), _BIG, jnp.float32)
        lane0 = iot == 0

        def lower_bound(target):
            def step(_, lohi):
                lo, hi = lohi
                mid = jnp.minimum(lax.shift_right_logical(lo + hi, 1), n - 1)
                bm = plsc.load_gather(batch_v, [mid])
                less = bm < target
                return (jnp.where(less, mid + 1, lo), jnp.where(less, hi, mid))
            lo, _ = lax.fori_loop(0, 12, step,
                                  (jnp.zeros((_LANES,), jnp.int32),
                                   jnp.full((_LANES,), n, jnp.int32)))
            return lo

        def make_d2(c0x, lenx, sqi, cxb, cyb, czb):
            def compute_d2(loc):
                gi = jnp.minimum(c0x + loc, n - 1)
                g3 = gi * 3
                xs = plsc.load_gather(posf_v, [g3])
                ys = plsc.load_gather(posf_v, [g3 + 1])
                zs = plsc.load_gather(posf_v, [g3 + 2])
                sqj = xs * xs + ys * ys + zs * zs
                dot = (cxb * _bf16_round(xs) + cyb * _bf16_round(ys)) \
                    + czb * _bf16_round(zs)
                d2 = jnp.maximum((sqi + sqj) - 2.0 * dot, 0.0)
                keep = (loc < lenx) & (d2 <= _CUTOFF2)
                return jnp.where(keep, d2, _BIG)
            return compute_d2

        def emit(sel, val, half, i, off, cx, cy, cz):
            src = jnp.where(val < _BIGH, sel, i)
            s3 = src * 3
            xs = plsc.load_gather(posf_v, [s3])
            ys = plsc.load_gather(posf_v, [s3 + 1])
            zs = plsc.load_gather(posf_v, [s3 + 2])
            vx = xs - cx
            vy = ys - cy
            vz = zs - cz
            d2e = vx * vx + vy * vy + vz * vz
            nonself = src != i
            w = jnp.where(nonself, d2e * _rsqrt_newton(d2e), 0.0)
            e = off + half * _LANES + iot
            zero = jnp.zeros((_LANES,), jnp.int32)
            plsc.store_scatter(src_v, [e], src)
            plsc.store_scatter(dst_v, [e], zero + i)
            plsc.store_scatter(wgt_v, [e], w)
            e3 = e * 3
            plsc.store_scatter(vec_v, [e3], vx)
            plsc.store_scatter(vec_v, [e3 + 1], vy)
            plsc.store_scatter(vec_v, [e3 + 2], vz)

        def group_body(g, _):
            rows = rowbase + g * _LANES + iot
            bv = plsc.load_gather(batch_v, [rows])
            c0g = lower_bound(bv)
            leng = lower_bound(bv + 1) - c0g
            r3 = rows * 3
            cxg = plsc.load_gather(posf_v, [r3])
            cyg = plsc.load_gather(posf_v, [r3 + 1])
            czg = plsc.load_gather(posf_v, [r3 + 2])
            gmax = jnp.max(leng)

            def row_common(r, fast_only):
                i = rowbase + g * _LANES + r
                rs = jnp.zeros((_LANES,), jnp.int32) + r
                c0v = _shuffle(c0g, rs)
                lenv = _shuffle(leng, rs)
                cx = _shuffle(cxg, rs)
                cy = _shuffle(cyg, rs)
                cz = _shuffle(czg, rs)
                sqi = cx * cx + cy * cy + cz * cz
                cxb = _bf16_round(cx)
                cyb = _bf16_round(cy)
                czb = _bf16_round(cz)
                d2v = make_d2(c0v, lenv, sqi, cxb, cyb, czb)
                off = (i - rowbase) * _K

                def fast_path(nv):
                    P = [(d2v(t * _LANES + iot), c0v + t * _LANES + iot)
                         for t in range(nv)]
                    return _bitonic_top32(P, iot)

                if fast_only:
                    s0, s1, v0, v1 = fast_path(4)
                    emit(s0, v0, 0, i, off, cx, cy, cz)
                    emit(s1, v1, 1, i, off, cx, cy, cz)
                    return

                c0 = jnp.max(c0v)
                seglen = jnp.max(lenv)
                nch = lax.shift_right_logical(seglen + (_LANES - 1), 4)
                d2s = make_d2(c0, seglen, sqi, cxb, cyb, czb)

                def slow_path(_):
                    def fill(t, _):
                        loc = t * _LANES + iot
                        plsc.store_scatter(d2buf, [loc], d2s(loc))
                        return 0
                    lax.fori_loop(0, nch, fill, 0)

                    def sel_cond(c):
                        kk, last = c[0], c[1]
                        return (kk < _K) & (last < _BIGH)

                    def sel_body(c):
                        kk, _, s0, s1, v0, v1 = c

                        def scan(t, mi):
                            mv, ivv = mi
                            loc = t * _LANES + iot
                            v = plsc.load_gather(d2buf, [loc])
                            lt = v < mv
                            return (jnp.where(lt, v, mv),
                                    jnp.where(lt, loc, ivv))
                        mv, ivv = lax.fori_loop(
                            0, nch, scan,
                            (jnp.full((_LANES,), _INIT, jnp.float32),
                             jnp.full((_LANES,), _MAXI, jnp.int32)))
                        m = jnp.min(mv)
                        loc = jnp.min(jnp.where(mv == m, ivv, _MAXI))
                        plsc.store_scatter(
                            d2buf, [jnp.full((_LANES,), loc, jnp.int32)],
                            bigv, mask=lane0)
                        gidx = c0 + loc
                        s0 = jnp.where(iot == kk, gidx, s0)
                        s1 = jnp.where(iot == kk - _LANES, gidx, s1)
                        v0 = jnp.where(iot == kk, m, v0)
                        v1 = jnp.where(iot == kk - _LANES, m, v1)
                        return (kk + 1, m, s0, s1, v0, v1)

                    centerv = jnp.full((_LANES,), 0, jnp.int32) + i
                    _, _, s0, s1, v0, v1 = lax.while_loop(
                        sel_cond, sel_body,
                        (0, 0.0, centerv, centerv, bigv, bigv))
                    return (s0, s1, v0, v1)

                s0, s1, v0, v1 = lax.cond(
                    seglen <= 2 * _LANES,
                    lambda _: fast_path(2),
                    lambda _: lax.cond(seglen <= 4 * _LANES,
                                       lambda _: fast_path(4), slow_path, 0),
                    0)
                emit(s0, v0, 0, i, off, cx, cy, cz)
                emit(s1, v1, 1, i, off, cx, cy, cz)

            def fast_rows(_):
                @plsc.parallel_loop(0, _LANES, 1, unroll=4)
                def frow(r):
                    row_common(r, True)
                return 0

            def gen_rows(_):
                def row_body(r, _):
                    row_common(r, False)
                    return 0
                lax.fori_loop(0, _LANES, row_body, 0)
                return 0

            lax.cond(gmax <= 4 * _LANES, fast_rows, gen_rows, 0)
            return 0

        lax.fori_loop(0, rows_per_worker // _LANES, group_body, 0)

        eb = rowbase * _K
        cnt = rows_per_worker * _K
        pltpu.sync_copy(src_v, ei_hbm.at[0, pl.ds(eb, cnt)])
        pltpu.sync_copy(dst_v, ei_hbm.at[1, pl.ds(eb, cnt)])
        pltpu.sync_copy(wgt_v, wgt_hbm.at[pl.ds(eb, cnt)])
        pltpu.sync_copy(vec_v, vec_hbm.at[pl.ds(eb * 3, cnt * 3)])

    return body(posf, batch)


def kernel(pos, batch):
    n = pos.shape[0]
    nw = _NC * _NS
    rows_per_worker = n // nw
    posf = pos.reshape(-1)
    ei, wgt, vecf = _build(posf, batch.astype(jnp.int32), n, rows_per_worker)
    return ei, wgt, vecf.reshape(n * _K, 3)


# speedup vs baseline: 1.0159x; 1.0159x over previous
import functools

import jax
import jax.numpy as jnp
from jax import lax
from jax.experimental import pallas as pl
from jax.experimental.pallas import tpu as pltpu
from jax.experimental.pallas import tpu_sc as plsc

_CUTOFF2 = 25.0
_K = 32
_BIG = 1e10
_BIGH = 0.5e10
_INIT = 2e10
_MAXI = 0x7FFFFFFF
_NC = 2
_NS = 16
_LANES = 16


def _iota():
    return lax.iota(jnp.int32, _LANES)


def _shuffle(v, perm):
    dnums = lax.GatherDimensionNumbers(offset_dims=(),
                                       collapsed_slice_dims=(0,),
                                       start_index_map=(0,))
    return lax.gather(v, perm[:, None], dnums, (1,),
                      mode=lax.GatherScatterMode.PROMISE_IN_BOUNDS)


def _bf16_round(x):
    b = plsc.bitcast(x, jnp.int32)
    r = b + 0x7FFF + (lax.shift_right_arithmetic(b, 16) & 1)
    return plsc.bitcast(r & jnp.int32(-65536), jnp.float32)


def _rsqrt_newton(x):
    ib = plsc.bitcast(x, jnp.int32)
    y = plsc.bitcast(jnp.int32(0x5F3759DF) - lax.shift_right_arithmetic(ib, 1),
                     jnp.float32)
    for _ in range(2):
        y = y * (1.5 - 0.5 * x * y * y)
    return y


def _lexlt(va, ia, vb, ib):
    return (va < vb) | ((va == vb) & (ia < ib))


def _ce_cross(a, b, desc):
    va, ia = a
    vb, ib = b
    lt = _lexlt(va, ia, vb, ib)
    keep = ~lt if desc else lt
    return ((jnp.where(keep, va, vb), jnp.where(keep, ia, ib)),
            (jnp.where(keep, vb, va), jnp.where(keep, ib, ia)))


def _ce_intra(p, t, j, k, iot):
    v, ix = p
    perm = iot ^ j
    pv = _shuffle(v, perm)
    pi = _shuffle(ix, perm)
    lt = _lexlt(v, ix, pv, pi)
    keep = lt ^ ((iot & j) != 0) ^ (((t * _LANES + iot) & k) != 0)
    return (jnp.where(keep, v, pv), jnp.where(keep, ix, pi))


def _bitonic_top32(P, iot):
    nv = len(P)
    for k in (2, 4, 8, 16):
        for j in (8, 4, 2, 1):
            if j < k:
                P = [_ce_intra(P[t], t, j, k, iot) for t in range(nv)]
    if nv == 2:
        P[0], P[1] = _ce_cross(P[0], P[1], False)
        for j in (8, 4, 2, 1):
            P = [_ce_intra(P[t], t, j, 32, iot) for t in range(2)]
        return (P[0][1], P[1][1], P[0][0], P[1][0])
    P[0], P[1] = _ce_cross(P[0], P[1], False)
    P[2], P[3] = _ce_cross(P[2], P[3], True)
    for j in (8, 4, 2, 1):
        P = [_ce_intra(P[t], t, j, 32, iot) for t in range(4)]
    P[0], P[2] = _ce_cross(P[0], P[2], False)
    P[1], P[3] = _ce_cross(P[1], P[3], False)
    P[0], P[1] = _ce_cross(P[0], P[1], False)
    for j in (8, 4, 2, 1):
        P = [_ce_intra(P[t], t, j, 64, iot) for t in range(2)] + P[2:]
    return (P[0][1], P[1][1], P[0][0], P[1][0])


def _build(posf, batch, n, rows_per_worker):
    k3 = 3 * _K
    mesh = plsc.VectorSubcoreMesh(core_axis_name="c", subcore_axis_name="s",
                                  num_cores=_NC, num_subcores=_NS)
    out_type = (
        jax.ShapeDtypeStruct((2, n * _K), jnp.int32),
        jax.ShapeDtypeStruct((n * _K,), jnp.float32),
        jax.ShapeDtypeStruct((n * k3,), jnp.float32),
    )
    scratch = (
        pltpu.VMEM((3 * n,), jnp.float32),
        pltpu.VMEM((n,), jnp.int32),
        pltpu.VMEM((n,), jnp.float32),
        pltpu.VMEM((rows_per_worker * _K,), jnp.int32),
        pltpu.VMEM((rows_per_worker * _K,), jnp.int32),
        pltpu.VMEM((rows_per_worker * _K,), jnp.float32),
        pltpu.VMEM((rows_per_worker * k3,), jnp.float32),
        pltpu.SemaphoreType.DMA,
        pltpu.SemaphoreType.DMA,
    )

    @functools.partial(pl.kernel, out_type=out_type, mesh=mesh,
                       scratch_types=scratch,
                       compiler_params=pltpu.CompilerParams(
                           needs_layout_passes=False))
    def body(posf_hbm, batch_hbm, ei_hbm, wgt_hbm, vec_hbm,
             posf_v, batch_v, d2buf, src_v, dst_v, wgt_v, vec_v,
             in_sem, out_sem):
        wid = lax.axis_index("s") * _NC + lax.axis_index("c")
        rowbase = wid * rows_per_worker
        cp_pos = pltpu.async_copy(posf_hbm, posf_v, in_sem)
        cp_bat = pltpu.async_copy(batch_hbm, batch_v, in_sem)
        cp_pos.wait()
        cp_bat.wait()

        iot = _iota()
        bigv = jnp.full((_LANES,), _BIG, jnp.float32)
        lane0 = iot == 0

        def lower_bound(target):
            def step(_, lohi):
                lo, hi = lohi
                mid = jnp.minimum(lax.shift_right_logical(lo + hi, 1), n - 1)
                bm = plsc.load_gather(batch_v, [mid])
                less = bm < target
                return (jnp.where(less, mid + 1, lo), jnp.where(less, hi, mid))
            lo, _ = lax.fori_loop(0, 12, step,
                                  (jnp.zeros((_LANES,), jnp.int32),
                                   jnp.full((_LANES,), n, jnp.int32)))
            return lo

        def make_d2(c0x, lenx, sqi, cxb, cyb, czb):
            def compute_d2(loc):
                gi = jnp.minimum(c0x + loc, n - 1)
                g3 = gi * 3
                xs = plsc.load_gather(posf_v, [g3])
                ys = plsc.load_gather(posf_v, [g3 + 1])
                zs = plsc.load_gather(posf_v, [g3 + 2])
                sqj = xs * xs + ys * ys + zs * zs
                dot = (cxb * _bf16_round(xs) + cyb * _bf16_round(ys)) \
                    + czb * _bf16_round(zs)
                d2 = jnp.maximum((sqi + sqj) - 2.0 * dot, 0.0)
                keep = (loc < lenx) & (d2 <= _CUTOFF2)
                return jnp.where(keep, d2, _BIG)
            return compute_d2

        def emit(sel, val, half, i, off, cx, cy, cz):
            src = jnp.where(val < _BIGH, sel, i)
            s3 = src * 3
            xs = plsc.load_gather(posf_v, [s3])
            ys = plsc.load_gather(posf_v, [s3 + 1])
            zs = plsc.load_gather(posf_v, [s3 + 2])
            vx = xs - cx
            vy = ys - cy
            vz = zs - cz
            d2e = vx * vx + vy * vy + vz * vz
            nonself = src != i
            w = jnp.where(nonself, d2e * _rsqrt_newton(d2e), 0.0)
            e = off + half * _LANES + iot
            zero = jnp.zeros((_LANES,), jnp.int32)
            plsc.store_scatter(src_v, [e], src)
            plsc.store_scatter(dst_v, [e], zero + i)
            plsc.store_scatter(wgt_v, [e], w)
            e3 = e * 3
            plsc.store_scatter(vec_v, [e3], vx)
            plsc.store_scatter(vec_v, [e3 + 1], vy)
            plsc.store_scatter(vec_v, [e3 + 2], vz)

        def group_body(g, _):
            rows = rowbase + g * _LANES + iot
            bv = plsc.load_gather(batch_v, [rows])
            c0g = lower_bound(bv)
            leng = lower_bound(bv + 1) - c0g
            r3 = rows * 3
            cxg = plsc.load_gather(posf_v, [r3])
            cyg = plsc.load_gather(posf_v, [r3 + 1])
            czg = plsc.load_gather(posf_v, [r3 + 2])
            gmax = jnp.max(leng)

            def row_common(r, fast_only):
                i = rowbase + g * _LANES + r
                rs = jnp.zeros((_LANES,), jnp.int32) + r
                c0v = _shuffle(c0g, rs)
                lenv = _shuffle(leng, rs)
                cx = _shuffle(cxg, rs)
                cy = _shuffle(cyg, rs)
                cz = _shuffle(czg, rs)
                sqi = cx * cx + cy * cy + cz * cz
                cxb = _bf16_round(cx)
                cyb = _bf16_round(cy)
                czb = _bf16_round(cz)
                d2v = make_d2(c0v, lenv, sqi, cxb, cyb, czb)
                off = (i - rowbase) * _K

                def fast_path(nv):
                    P = [(d2v(t * _LANES + iot), c0v + t * _LANES + iot)
                         for t in range(nv)]
                    return _bitonic_top32(P, iot)

                if fast_only:
                    s0, s1, v0, v1 = fast_path(4)
                    emit(s0, v0, 0, i, off, cx, cy, cz)
                    emit(s1, v1, 1, i, off, cx, cy, cz)
                    return

                c0 = jnp.max(c0v)
                seglen = jnp.max(lenv)
                nch = lax.shift_right_logical(seglen + (_LANES - 1), 4)
                d2s = make_d2(c0, seglen, sqi, cxb, cyb, czb)

                def slow_path(_):
                    def fill(t, _):
                        loc = t * _LANES + iot
                        plsc.store_scatter(d2buf, [loc], d2s(loc))
                        return 0
                    lax.fori_loop(0, nch, fill, 0)

                    def sel_cond(c):
                        kk, last = c[0], c[1]
                        return (kk < _K) & (last < _BIGH)

                    def sel_body(c):
                        kk, _, s0, s1, v0, v1 = c

                        def scan(t, mi):
                            mv, ivv = mi
                            loc = t * _LANES + iot
                            v = plsc.load_gather(d2buf, [loc])
                            lt = v < mv
                            return (jnp.where(lt, v, mv),
                                    jnp.where(lt, loc, ivv))
                        mv, ivv = lax.fori_loop(
                            0, nch, scan,
                            (jnp.full((_LANES,), _INIT, jnp.float32),
                             jnp.full((_LANES,), _MAXI, jnp.int32)))
                        m = jnp.min(mv)
                        loc = jnp.min(jnp.where(mv == m, ivv, _MAXI))
                        plsc.store_scatter(
                            d2buf, [jnp.full((_LANES,), loc, jnp.int32)],
                            bigv, mask=lane0)
                        gidx = c0 + loc
                        s0 = jnp.where(iot == kk, gidx, s0)
                        s1 = jnp.where(iot == kk - _LANES, gidx, s1)
                        v0 = jnp.where(iot == kk, m, v0)
                        v1 = jnp.where(iot == kk - _LANES, m, v1)
                        return (kk + 1, m, s0, s1, v0, v1)

                    centerv = jnp.full((_LANES,), 0, jnp.int32) + i
                    _, _, s0, s1, v0, v1 = lax.while_loop(
                        sel_cond, sel_body,
                        (0, 0.0, centerv, centerv, bigv, bigv))
                    return (s0, s1, v0, v1)

                s0, s1, v0, v1 = lax.cond(
                    seglen <= 2 * _LANES,
                    lambda _: fast_path(2),
                    lambda _: lax.cond(seglen <= 4 * _LANES,
                                       lambda _: fast_path(4), slow_path, 0),
                    0)
                emit(s0, v0, 0, i, off, cx, cy, cz)
                emit(s1, v1, 1, i, off, cx, cy, cz)

            def fast_rows(_):
                @plsc.parallel_loop(0, _LANES, 1, unroll=2)
                def frow(r):
                    row_common(r, True)
                return 0

            def gen_rows(_):
                def row_body(r, _):
                    row_common(r, False)
                    return 0
                lax.fori_loop(0, _LANES, row_body, 0)
                return 0

            lax.cond(gmax <= 4 * _LANES, fast_rows, gen_rows, 0)
            return 0

        lax.fori_loop(0, rows_per_worker // _LANES, group_body, 0)

        eb = rowbase * _K
        cnt = rows_per_worker * _K
        o1 = pltpu.async_copy(src_v, ei_hbm.at[0, pl.ds(eb, cnt)], out_sem)
        o2 = pltpu.async_copy(dst_v, ei_hbm.at[1, pl.ds(eb, cnt)], out_sem)
        o3 = pltpu.async_copy(wgt_v, wgt_hbm.at[pl.ds(eb, cnt)], out_sem)
        o4 = pltpu.async_copy(vec_v, vec_hbm.at[pl.ds(eb * 3, cnt * 3)],
                              out_sem)
        o1.wait()
        o2.wait()
        o3.wait()
        o4.wait()

    return body(posf, batch)


def kernel(pos, batch):
    n = pos.shape[0]
    nw = _NC * _NS
    rows_per_worker = n // nw
    posf = pos.reshape(-1)
    ei, wgt, vecf = _build(posf, batch.astype(jnp.int32), n, rows_per_worker)
    return ei, wgt, vecf.reshape(n * _K, 3)
